# baseline (speedup 1.0000x reference)
import math
import functools

import jax
import jax.numpy as jnp
import numpy as np
from jax import lax
from jax.experimental import pallas as pl
from jax.experimental.pallas import tpu as pltpu
from jax.experimental.pallas import tpu_sc as plsc

_VOCAB = 100000
_D = 64
_CHUNK = 32
_BATCH = 1024
_SEQ = 512
_NCHUNKS = _SEQ // _CHUNK
_SCALE = math.sqrt(_D)

_NC, _NS, _L = 2, 16, 16
_NW = _NC * _NS
_KL = _D // _L

_N_IDS = _BATCH * _SEQ
_IDS_PER_W = _N_IDS // _NW
_CPW = _IDS_PER_W // _CHUNK
_GIDX = 128
_CPG = _GIDX // _CHUNK
_NG = _IDS_PER_W // _GIDX
_NBUF = 4

_OUT_PER_W = _CPW * _D


def _pe_chunk_mean():
    position = np.arange(_SEQ, dtype=np.float32)[:, None]
    div_term = np.exp(
        np.arange(0, _D, 2, dtype=np.float32) * (-math.log(10000.0) / _D))
    pe = np.zeros((_SEQ, _D), dtype=np.float32)
    pe[:, 0::2] = np.sin(position * div_term)
    pe[:, 1::2] = np.cos(position * div_term)
    return pe.reshape(_NCHUNKS, _CHUNK, _D).mean(axis=1)


_PE_MEAN = _pe_chunk_mean().reshape(-1)


def _sc_body(ids_hbm, table_hbm, pe_hbm, out_hbm,
             idx_v, rows0, rows1, rows2, rows3,
             out_v, pe_v,
             sem0, sem1, sem2, sem3):
    rows = (rows0, rows1, rows2, rows3)
    sems = (sem0, sem1, sem2, sem3)
    wid = lax.axis_index("s") * _NC + lax.axis_index("c")

    pltpu.sync_copy(ids_hbm.at[pl.ds(wid * _IDS_PER_W, _IDS_PER_W)], idx_v)
    pltpu.sync_copy(pe_hbm, pe_v)

    def _gather(g, b):
        idx = idx_v.at[pl.ds(lax.mul(g, _GIDX), _GIDX)]
        return pltpu.make_async_copy(table_hbm.at[idx], rows[b], sems[b])

    for b in range(_NBUF):
        _gather(b, b).start()

    @pl.loop(0, _NG, step=_NBUF)
    def _group_loop(gg):
        for b in range(_NBUF):
            g = gg + b
            _gather(g, b).wait()

            @pl.loop(0, _CPG, unroll=4)
            def _chunk(j):
                base = lax.mul(j, _CHUNK)
                accs = [rows[b][base, pl.ds(k * _L, _L)] for k in range(_KL)]
                for r in range(1, _CHUNK):
                    for k in range(_KL):
                        accs[k] = accs[k] + rows[b][base + r, pl.ds(k * _L, _L)]
                crow = lax.mul(g, _CPG) + j
                obase = lax.mul(crow, _D)
                pbase = lax.mul(lax.rem(crow, _NCHUNKS), _D)
                for k in range(_KL):
                    out_v[pl.ds(obase + k * _L, _L)] = (
                        accs[k] * (_SCALE / _CHUNK)
                        + pe_v[pl.ds(pbase + k * _L, _L)])

            @pl.when(g + _NBUF < _NG)
            def _():
                _gather(g + _NBUF, b).start()

    pltpu.sync_copy(out_v, out_hbm.at[pl.ds(wid * _OUT_PER_W, _OUT_PER_W)])


@functools.cache
def _sc_call():
  return pl.kernel(
    _sc_body,
    out_type=jax.ShapeDtypeStruct((_N_IDS * _D // _CHUNK,), jnp.float32),
    mesh=plsc.VectorSubcoreMesh(core_axis_name="c", subcore_axis_name="s",
                                num_cores=_NC, num_subcores=_NS),
    scratch_types=[
        pltpu.VMEM((_IDS_PER_W,), jnp.int32),
        pltpu.VMEM((_GIDX, _D), jnp.float32),
        pltpu.VMEM((_GIDX, _D), jnp.float32),
        pltpu.VMEM((_GIDX, _D), jnp.float32),
        pltpu.VMEM((_GIDX, _D), jnp.float32),
        pltpu.VMEM((_OUT_PER_W,), jnp.float32),
        pltpu.VMEM((_NCHUNKS * _D,), jnp.float32),
        pltpu.SemaphoreType.DMA,
        pltpu.SemaphoreType.DMA,
        pltpu.SemaphoreType.DMA,
        pltpu.SemaphoreType.DMA,
    ],
    compiler_params=pltpu.CompilerParams(use_tc_tiling_on_sc=False),
  )


@jax.jit
def kernel(token_ids, embedding):
    ids = token_ids.astype(jnp.int32).reshape(-1)
    pe = jnp.asarray(_PE_MEAN)
    out = _sc_call()(ids, embedding, pe)
    return out.reshape(_BATCH, _NCHUNKS, _D)


# speedup vs baseline: 1.4239x; 1.4239x over previous
import math
import functools

import jax
import jax.numpy as jnp
import numpy as np
from jax import lax
from jax.experimental import pallas as pl
from jax.experimental.pallas import tpu as pltpu
from jax.experimental.pallas import tpu_sc as plsc

_VOCAB = 100000
_D = 64
_CHUNK = 32
_BATCH = 1024
_SEQ = 512
_NCHUNKS = _SEQ // _CHUNK
_SCALE = math.sqrt(_D)

_NC, _NS, _L = 2, 16, 16
_NW = _NC * _NS
_KL = _D // _L

_N_IDS = _BATCH * _SEQ
_IDS_PER_W = _N_IDS // _NW
_CPW = _IDS_PER_W // _CHUNK
_GIDX = 128
_CPG = _GIDX // _CHUNK
_NG = _IDS_PER_W // _GIDX
_NBUF = 4

_OUT_PER_W = _CPW * _D


def _pe_chunk_mean():
    position = np.arange(_SEQ, dtype=np.float32)[:, None]
    div_term = np.exp(
        np.arange(0, _D, 2, dtype=np.float32) * (-math.log(10000.0) / _D))
    pe = np.zeros((_SEQ, _D), dtype=np.float32)
    pe[:, 0::2] = np.sin(position * div_term)
    pe[:, 1::2] = np.cos(position * div_term)
    return pe.reshape(_NCHUNKS, _CHUNK, _D).mean(axis=1)


_PE_MEAN = _pe_chunk_mean().reshape(-1)


def _sc_body(ids_hbm, table_hbm, pe_hbm, out_hbm,
             idx_v, rows0, rows1, rows2, rows3,
             out_v, pe_v,
             sem0, sem1, sem2, sem3):
    rows = (rows0, rows1, rows2, rows3)
    sems = (sem0, sem1, sem2, sem3)
    wid = lax.axis_index("s") * _NC + lax.axis_index("c")

    pltpu.sync_copy(ids_hbm.at[pl.ds(wid * _IDS_PER_W, _IDS_PER_W)], idx_v)
    pltpu.sync_copy(pe_hbm, pe_v)

    def _gather(g, b):
        idx = idx_v.at[pl.ds(lax.mul(g, _GIDX), _GIDX)]
        return pltpu.make_async_copy(table_hbm.at[idx], rows[b], sems[b])

    for b in range(_NBUF):
        _gather(b, b).start()

    @pl.loop(0, _NG, step=_NBUF)
    def _group_loop(gg):
        for b in range(_NBUF):
            g = gg + b
            _gather(g, b).wait()

            @pl.loop(0, _CPG, unroll=2)
            def _chunk(j):
                base = lax.mul(j, _CHUNK)
                accs = [rows[b][base, pl.ds(k * _L, _L)] for k in range(_KL)]
                for r in range(1, _CHUNK):
                    for k in range(_KL):
                        accs[k] = accs[k] + rows[b][base + r, pl.ds(k * _L, _L)]
                crow = lax.mul(g, _CPG) + j
                obase = lax.mul(crow, _D)
                pbase = lax.mul(lax.rem(crow, _NCHUNKS), _D)
                for k in range(_KL):
                    out_v[pl.ds(obase + k * _L, _L)] = (
                        accs[k] * (_SCALE / _CHUNK)
                        + pe_v[pl.ds(pbase + k * _L, _L)])

            @pl.when(g + _NBUF < _NG)
            def _():
                _gather(g + _NBUF, b).start()

    pltpu.sync_copy(out_v, out_hbm.at[pl.ds(wid * _OUT_PER_W, _OUT_PER_W)])


@functools.cache
def _sc_call():
  return pl.kernel(
    _sc_body,
    out_type=jax.ShapeDtypeStruct((_N_IDS * _D // _CHUNK,), jnp.float32),
    mesh=plsc.VectorSubcoreMesh(core_axis_name="c", subcore_axis_name="s",
                                num_cores=_NC, num_subcores=_NS),
    scratch_types=[
        pltpu.VMEM((_IDS_PER_W,), jnp.int32),
        pltpu.VMEM((_GIDX, _D), jnp.float32),
        pltpu.VMEM((_GIDX, _D), jnp.float32),
        pltpu.VMEM((_GIDX, _D), jnp.float32),
        pltpu.VMEM((_GIDX, _D), jnp.float32),
        pltpu.VMEM((_OUT_PER_W,), jnp.float32),
        pltpu.VMEM((_NCHUNKS * _D,), jnp.float32),
        pltpu.SemaphoreType.DMA,
        pltpu.SemaphoreType.DMA,
        pltpu.SemaphoreType.DMA,
        pltpu.SemaphoreType.DMA,
    ],
    compiler_params=pltpu.CompilerParams(use_tc_tiling_on_sc=False),
  )


@jax.jit
def kernel(token_ids, embedding):
    ids = token_ids.astype(jnp.int32).reshape(-1)
    pe = jnp.asarray(_PE_MEAN)
    out = _sc_call()(ids, embedding, pe)
    return out.reshape(_BATCH, _NCHUNKS, _D)
